# baseline (speedup 1.0000x reference)
import functools

import jax
import jax.numpy as jnp
from jax.experimental import pallas as pl
from jax.experimental.pallas import tpu as pltpu

_C = 80
_THIRD = 1.0 / 3.0


def _tc_body(nblk, cls_ref, aux_ref, aux4_ref, pb_ref, tb_ref,
             out_cls_ref, out_reg_ref, cvec, r_acc, n_acc):
    i = pl.program_id(0)

    @pl.when(i == 0)
    def _init():
        cvec[...] = jnp.zeros_like(cvec)
        r_acc[0] = 0.0
        n_acc[0] = 0.0

    x = cls_ref[...]
    tcmp = aux_ref[...][:, 0:1]
    validf = aux_ref[...][:, 1:2]

    e2 = jnp.exp(-x)
    a = 1.0 + e2
    p = 1.0 / a
    lg = jnp.log(a)
    sp = x + lg
    omp = e2 * p
    l0 = sp * p * p
    l1 = _THIRD * lg * omp * omp
    cls_iota = jax.lax.broadcasted_iota(jnp.int32, x.shape, 1).astype(jnp.float32)
    fl = jnp.where(cls_iota == tcmp, l1, l0) * validf
    cvec[...] += jnp.sum(fl, axis=0)[None, :]

    def roll(v, s):
        return pltpu.roll(v, v.shape[1] - s, 1)

    b1 = pb_ref[0]
    b2 = tb_ref[0]
    lane4 = jax.lax.broadcasted_iota(jnp.int32, b1.shape, 1) % 4
    is01 = lane4 < 2
    ltrb = jnp.where(is01, jnp.maximum(b1, b2), jnp.minimum(b1, b2))
    d = jnp.maximum(roll(ltrb, 2) - ltrb, 0.0)
    inter = d * roll(d, 1)
    d1 = roll(b1, 2) - b1
    area1 = d1 * roll(d1, 1)
    d2 = roll(b2, 2) - b2
    area2 = d2 * roll(d2, 1)
    union = area1 + area2 - inter
    ltrbc = jnp.where(is01, jnp.minimum(b1, b2), jnp.maximum(b1, b2))
    dc = jnp.maximum(roll(ltrbc, 2) - ltrbc, 0.0)
    areac = dc * roll(dc, 1)
    giou = inter / union - (areac - union) / areac
    live = (lane4 == 0) & (aux4_ref[0] >= 0.0)
    r_acc[0] += jnp.sum(jnp.where(live, 1.0 - giou, 0.0))
    n_acc[0] += jnp.sum(jnp.where(live, 1.0, 0.0))

    @pl.when(i == nblk - 1)
    def _fin():
        denom = jnp.maximum(n_acc[0], 1.0)
        out_cls_ref[...] = jnp.full((1, 1), 0.75 * jnp.sum(cvec[...]) / denom,
                                    jnp.float32)
        out_reg_ref[...] = jnp.full((1, 1), r_acc[0] / denom, jnp.float32)


def kernel(pred_cls, pred_box, mask, cls_targets, box_targets):
    B, M, C = pred_cls.shape
    N = B * M
    BLK = 2048
    BLK4 = BLK // 8 * 4
    G = N // BLK

    x = pred_cls.reshape(N, C)
    pb = pred_box.reshape(G, 8, BLK4)
    tb = box_targets.reshape(G, 8, BLK4)
    t = cls_targets.astype(jnp.float32)
    fg = (cls_targets >= 0) & (cls_targets != C)
    valid = (cls_targets >= 0) & jnp.logical_not(mask.reshape(N))
    tcmp = jnp.where(fg, t, -1.0)
    validf = valid.astype(jnp.float32)
    aux = jnp.stack([tcmp, validf, tcmp, validf], axis=1)
    aux4 = aux.reshape(G, 8, BLK4)

    out_cls, out_reg = pl.pallas_call(
        functools.partial(_tc_body, G),
        grid=(G,),
        in_specs=[
            pl.BlockSpec((BLK, C), lambda i: (i, 0)),
            pl.BlockSpec((BLK, 4), lambda i: (i, 0)),
            pl.BlockSpec((1, 8, BLK4), lambda i: (i, 0, 0)),
            pl.BlockSpec((1, 8, BLK4), lambda i: (i, 0, 0)),
            pl.BlockSpec((1, 8, BLK4), lambda i: (i, 0, 0)),
        ],
        out_specs=[
            pl.BlockSpec((1, 1), lambda i: (0, 0)),
            pl.BlockSpec((1, 1), lambda i: (0, 0)),
        ],
        out_shape=[
            jax.ShapeDtypeStruct((1, 1), jnp.float32),
            jax.ShapeDtypeStruct((1, 1), jnp.float32),
        ],
        scratch_shapes=[
            pltpu.VMEM((1, C), jnp.float32),
            pltpu.SMEM((1,), jnp.float32),
            pltpu.SMEM((1,), jnp.float32),
        ],
    )(x, aux, aux4, pb, tb)
    return (out_cls[0, 0], out_reg[0, 0])


# speedup vs baseline: 1.0116x; 1.0116x over previous
import functools

import jax
import jax.numpy as jnp
from jax.experimental import pallas as pl
from jax.experimental.pallas import tpu as pltpu

_C = 80
_THIRD = 1.0 / 3.0


def _tc_body(nblk, cls_ref, aux_ref, aux4_ref, pb_ref, tb_ref,
             out_cls_ref, out_reg_ref, cvec, r_acc, n_acc):
    i = pl.program_id(0)

    @pl.when(i == 0)
    def _init():
        cvec[...] = jnp.zeros_like(cvec)
        r_acc[0] = 0.0
        n_acc[0] = 0.0

    x = cls_ref[...]
    tcmp = aux_ref[...][:, 0:1]
    validf = aux_ref[...][:, 1:2]

    e2 = jnp.exp(-x)
    a = 1.0 + e2
    p = 1.0 / a
    lg = jnp.log(a)
    sp = x + lg
    omp = e2 * p
    l0 = sp * p * p
    l1 = _THIRD * lg * omp * omp
    cls_iota = jax.lax.broadcasted_iota(jnp.int32, x.shape, 1).astype(jnp.float32)
    fl = jnp.where(cls_iota == tcmp, l1, l0) * validf
    cvec[...] += jnp.sum(fl, axis=0)[None, :]

    def roll(v, s):
        return pltpu.roll(v, v.shape[1] - s, 1)

    b1 = pb_ref[...]
    b2 = tb_ref[...]
    lane4 = jax.lax.broadcasted_iota(jnp.int32, b1.shape, 1) % 4
    is01 = lane4 < 2
    mn = jnp.minimum(b1, b2)
    mx = jnp.maximum(b1, b2)
    ltrb = jnp.where(is01, mx, mn)
    d = jnp.maximum(roll(ltrb, 2) - ltrb, 0.0)
    inter = d * roll(d, 1)
    d1 = roll(b1, 2) - b1
    area1 = d1 * roll(d1, 1)
    d2 = roll(b2, 2) - b2
    area2 = d2 * roll(d2, 1)
    union = area1 + area2 - inter
    ltrbc = jnp.where(is01, mn, mx)
    dc = jnp.maximum(roll(ltrbc, 2) - ltrbc, 0.0)
    areac = dc * roll(dc, 1)
    giou = inter / union - (areac - union) / areac
    live = (lane4 == 0) & (aux4_ref[...] >= 0.0)
    r_acc[0] += jnp.sum(jnp.where(live, 1.0 - giou, 0.0))
    n_acc[0] += jnp.sum(jnp.where(live, 1.0, 0.0))

    @pl.when(i == nblk - 1)
    def _fin():
        denom = jnp.maximum(n_acc[0], 1.0)
        out_cls_ref[...] = jnp.full((1, 1), 0.75 * jnp.sum(cvec[...]) / denom,
                                    jnp.float32)
        out_reg_ref[...] = jnp.full((1, 1), r_acc[0] / denom, jnp.float32)


def kernel(pred_cls, pred_box, mask, cls_targets, box_targets):
    B, M, C = pred_cls.shape
    N = B * M
    BLK = 2048
    BLK4 = BLK // 32
    G = N // BLK

    x = pred_cls.reshape(N, C)
    pb = pred_box.reshape(N // 32, 128)
    tb = box_targets.reshape(N // 32, 128)
    t = cls_targets.astype(jnp.float32)
    fg = (cls_targets >= 0) & (cls_targets != C)
    valid = (cls_targets >= 0) & jnp.logical_not(mask.reshape(N))
    tcmp = jnp.where(fg, t, -1.0)
    validf = valid.astype(jnp.float32)
    aux = jnp.stack([tcmp, validf, tcmp, validf], axis=1)
    aux4 = aux.reshape(N // 32, 128)

    out_cls, out_reg = pl.pallas_call(
        functools.partial(_tc_body, G),
        grid=(G,),
        in_specs=[
            pl.BlockSpec((BLK, C), lambda i: (i, 0)),
            pl.BlockSpec((BLK, 4), lambda i: (i, 0)),
            pl.BlockSpec((BLK4, 128), lambda i: (i, 0)),
            pl.BlockSpec((BLK4, 128), lambda i: (i, 0)),
            pl.BlockSpec((BLK4, 128), lambda i: (i, 0)),
        ],
        out_specs=[
            pl.BlockSpec((1, 1), lambda i: (0, 0)),
            pl.BlockSpec((1, 1), lambda i: (0, 0)),
        ],
        out_shape=[
            jax.ShapeDtypeStruct((1, 1), jnp.float32),
            jax.ShapeDtypeStruct((1, 1), jnp.float32),
        ],
        scratch_shapes=[
            pltpu.VMEM((1, C), jnp.float32),
            pltpu.SMEM((1,), jnp.float32),
            pltpu.SMEM((1,), jnp.float32),
        ],
    )(x, aux, aux4, pb, tb)
    return (out_cls[0, 0], out_reg[0, 0])


# speedup vs baseline: 1.7244x; 1.7046x over previous
import functools

import jax
import jax.numpy as jnp
from jax.experimental import pallas as pl
from jax.experimental.pallas import tpu as pltpu

_C = 80
_THIRD = 1.0 / 3.0
_SUB = 16


def _tc_body(nblk, cls_ref, tc_ref, vc_ref, tl_ref, pb_ref, tb_ref,
             out_cls_ref, out_reg_ref, cvec, r_acc, n_acc):
    i = pl.program_id(0)

    @pl.when(i == 0)
    def _init():
        cvec[...] = jnp.zeros_like(cvec)
        r_acc[0] = 0.0
        n_acc[0] = 0.0

    x = cls_ref[0]
    tcol = tc_ref[0]
    vcol = vc_ref[0]

    e2 = jnp.exp(-x)
    a = 1.0 + e2
    p = 1.0 / a
    lg = jnp.log(a)
    sp = x + lg
    omp = e2 * p
    l0 = sp * p * p
    l1 = _THIRD * lg * omp * omp
    sub_iota = jax.lax.broadcasted_iota(
        jnp.int32, (128, _C), 1).astype(jnp.float32)
    csum = None
    for k in range(_SUB):
        tk = tcol[:, k:k + 1]
        vk = vcol[:, k:k + 1]
        l0k = l0[128 * k:128 * (k + 1), :]
        l1k = l1[128 * k:128 * (k + 1), :]
        flk = jnp.where(sub_iota == tk, l1k, l0k) * vk
        csum = flk if csum is None else csum + flk
    cvec[...] += jnp.sum(csum, axis=0)[None, :]

    b1 = jnp.transpose(pb_ref[0], (1, 0))
    b2 = jnp.transpose(tb_ref[0], (1, 0))
    tlane = tl_ref[0]
    rsum = None
    csel = None
    for k in range(_SUB):
        s1 = b1[:, 128 * k:128 * (k + 1)]
        s2 = b2[:, 128 * k:128 * (k + 1)]
        b1x0, b1y0, b1x1, b1y1 = s1[0:1], s1[1:2], s1[2:3], s1[3:4]
        b2x0, b2y0, b2x1, b2y1 = s2[0:1], s2[1:2], s2[2:3], s2[3:4]
        area1 = (b1x1 - b1x0) * (b1y1 - b1y0)
        area2 = (b2x1 - b2x0) * (b2y1 - b2y0)
        iw = jnp.maximum(jnp.minimum(b1x1, b2x1) - jnp.maximum(b1x0, b2x0), 0.0)
        ih = jnp.maximum(jnp.minimum(b1y1, b2y1) - jnp.maximum(b1y0, b2y0), 0.0)
        inter = iw * ih
        union = area1 + area2 - inter
        cw = jnp.maximum(jnp.maximum(b1x1, b2x1) - jnp.minimum(b1x0, b2x0), 0.0)
        ch = jnp.maximum(jnp.maximum(b1y1, b2y1) - jnp.minimum(b1y0, b2y0), 0.0)
        areac = cw * ch
        giou = inter / union - (areac - union) / areac
        fgk = tlane[k:k + 1, :] >= 0.0
        contrib = jnp.where(fgk, 1.0 - giou, 0.0)
        fsel = jnp.where(fgk, 1.0, 0.0)
        rsum = contrib if rsum is None else rsum + contrib
        csel = fsel if csel is None else csel + fsel
    r_acc[0] += jnp.sum(rsum)
    n_acc[0] += jnp.sum(csel)

    @pl.when(i == nblk - 1)
    def _fin():
        denom = jnp.maximum(n_acc[0], 1.0)
        out_cls_ref[...] = jnp.full((1, 1), 0.75 * jnp.sum(cvec[...]) / denom,
                                    jnp.float32)
        out_reg_ref[...] = jnp.full((1, 1), r_acc[0] / denom, jnp.float32)


def kernel(pred_cls, pred_box, mask, cls_targets, box_targets):
    B, M, C = pred_cls.shape
    N = B * M
    BLK = 2048
    G = N // BLK
    MB = M // BLK

    t = cls_targets.astype(jnp.float32)
    fg = (cls_targets >= 0) & (cls_targets != C)
    valid = (cls_targets >= 0) & jnp.logical_not(mask.reshape(N))
    tcmp = jnp.where(fg, t, -1.0)
    validf = valid.astype(jnp.float32)
    tc3 = tcmp.reshape(G, _SUB, 128).swapaxes(1, 2)
    vc3 = validf.reshape(G, _SUB, 128).swapaxes(1, 2)
    tl3 = tcmp.reshape(G, _SUB, 128)

    out_cls, out_reg = pl.pallas_call(
        functools.partial(_tc_body, G),
        grid=(G,),
        in_specs=[
            pl.BlockSpec((1, BLK, C), lambda i: (i // MB, i % MB, 0)),
            pl.BlockSpec((1, 128, _SUB), lambda i: (i, 0, 0)),
            pl.BlockSpec((1, 128, _SUB), lambda i: (i, 0, 0)),
            pl.BlockSpec((1, _SUB, 128), lambda i: (i, 0, 0)),
            pl.BlockSpec((1, BLK, 4), lambda i: (i // MB, i % MB, 0)),
            pl.BlockSpec((1, BLK, 4), lambda i: (i // MB, i % MB, 0)),
        ],
        out_specs=[
            pl.BlockSpec((1, 1), lambda i: (0, 0)),
            pl.BlockSpec((1, 1), lambda i: (0, 0)),
        ],
        out_shape=[
            jax.ShapeDtypeStruct((1, 1), jnp.float32),
            jax.ShapeDtypeStruct((1, 1), jnp.float32),
        ],
        scratch_shapes=[
            pltpu.VMEM((1, C), jnp.float32),
            pltpu.SMEM((1,), jnp.float32),
            pltpu.SMEM((1,), jnp.float32),
        ],
    )(pred_cls, tc3, vc3, tl3, pred_box,
      box_targets.reshape(B, M, 4))
    return (out_cls[0, 0], out_reg[0, 0])


# speedup vs baseline: 1.9669x; 1.1406x over previous
import functools

import jax
import jax.numpy as jnp
from jax.experimental import pallas as pl
from jax.experimental.pallas import tpu as pltpu

_C = 80
_THIRD = 1.0 / 3.0
_SUB = 64


def _tc_body(nblk, cls_ref, tc_ref, vc_ref, pb_ref, tb_ref,
             out_cls_ref, out_reg_ref, cvec, r_acc, n_acc):
    i = pl.program_id(0)

    @pl.when(i == 0)
    def _init():
        cvec[...] = jnp.zeros_like(cvec)
        r_acc[0] = 0.0
        n_acc[0] = 0.0

    x = cls_ref[0]
    tcol = tc_ref[0]
    vcol = vc_ref[0]

    e2 = jnp.exp(-x)
    a = 1.0 + e2
    p = 1.0 / a
    lg = jnp.log(a)
    sp = x + lg
    omp = e2 * p
    l0 = sp * p * p
    l1 = _THIRD * lg * omp * omp
    sub_iota = jax.lax.broadcasted_iota(
        jnp.int32, (128, _C), 1).astype(jnp.float32)
    csum = None
    for k in range(_SUB):
        tk = tcol[:, k:k + 1]
        vk = vcol[:, k:k + 1]
        l0k = l0[128 * k:128 * (k + 1), :]
        l1k = l1[128 * k:128 * (k + 1), :]
        flk = jnp.where(sub_iota == tk, l1k, l0k) * vk
        csum = flk if csum is None else csum + flk
    cvec[...] += jnp.sum(csum, axis=0)[None, :]

    b1 = jnp.transpose(pb_ref[0], (1, 0))
    b2 = jnp.transpose(tb_ref[0], (1, 0))
    tlane = jnp.transpose(tcol, (1, 0))
    rsum = None
    csel = None
    for k in range(_SUB):
        s1 = b1[:, 128 * k:128 * (k + 1)]
        s2 = b2[:, 128 * k:128 * (k + 1)]
        b1x0, b1y0, b1x1, b1y1 = s1[0:1], s1[1:2], s1[2:3], s1[3:4]
        b2x0, b2y0, b2x1, b2y1 = s2[0:1], s2[1:2], s2[2:3], s2[3:4]
        area1 = (b1x1 - b1x0) * (b1y1 - b1y0)
        area2 = (b2x1 - b2x0) * (b2y1 - b2y0)
        iw = jnp.maximum(jnp.minimum(b1x1, b2x1) - jnp.maximum(b1x0, b2x0), 0.0)
        ih = jnp.maximum(jnp.minimum(b1y1, b2y1) - jnp.maximum(b1y0, b2y0), 0.0)
        inter = iw * ih
        union = area1 + area2 - inter
        cw = jnp.maximum(jnp.maximum(b1x1, b2x1) - jnp.minimum(b1x0, b2x0), 0.0)
        ch = jnp.maximum(jnp.maximum(b1y1, b2y1) - jnp.minimum(b1y0, b2y0), 0.0)
        areac = cw * ch
        giou = inter / union - (areac - union) / areac
        fgk = tlane[k:k + 1, :] >= 0.0
        contrib = jnp.where(fgk, 1.0 - giou, 0.0)
        fsel = jnp.where(fgk, 1.0, 0.0)
        rsum = contrib if rsum is None else rsum + contrib
        csel = fsel if csel is None else csel + fsel
    r_acc[0] += jnp.sum(rsum)
    n_acc[0] += jnp.sum(csel)

    @pl.when(i == nblk - 1)
    def _fin():
        denom = jnp.maximum(n_acc[0], 1.0)
        out_cls_ref[...] = jnp.full((1, 1), 0.75 * jnp.sum(cvec[...]) / denom,
                                    jnp.float32)
        out_reg_ref[...] = jnp.full((1, 1), r_acc[0] / denom, jnp.float32)


def kernel(pred_cls, pred_box, mask, cls_targets, box_targets):
    B, M, C = pred_cls.shape
    N = B * M
    BLK = 8192
    G = N // BLK
    MB = M // BLK

    t = cls_targets.astype(jnp.float32)
    fg = (cls_targets >= 0) & (cls_targets != C)
    valid = (cls_targets >= 0) & jnp.logical_not(mask.reshape(N))
    tcmp = jnp.where(fg, t, -1.0)
    validf = valid.astype(jnp.float32)
    tc3 = tcmp.reshape(G, _SUB, 128).swapaxes(1, 2)
    vc3 = validf.reshape(G, _SUB, 128).swapaxes(1, 2)

    out_cls, out_reg = pl.pallas_call(
        functools.partial(_tc_body, G),
        grid=(G,),
        in_specs=[
            pl.BlockSpec((1, BLK, C), lambda i: (i // MB, i % MB, 0)),
            pl.BlockSpec((1, 128, _SUB), lambda i: (i, 0, 0)),
            pl.BlockSpec((1, 128, _SUB), lambda i: (i, 0, 0)),
            pl.BlockSpec((1, BLK, 4), lambda i: (i // MB, i % MB, 0)),
            pl.BlockSpec((1, BLK, 4), lambda i: (i // MB, i % MB, 0)),
        ],
        out_specs=[
            pl.BlockSpec((1, 1), lambda i: (0, 0)),
            pl.BlockSpec((1, 1), lambda i: (0, 0)),
        ],
        out_shape=[
            jax.ShapeDtypeStruct((1, 1), jnp.float32),
            jax.ShapeDtypeStruct((1, 1), jnp.float32),
        ],
        scratch_shapes=[
            pltpu.VMEM((1, C), jnp.float32),
            pltpu.SMEM((1,), jnp.float32),
            pltpu.SMEM((1,), jnp.float32),
        ],
    )(pred_cls, tc3, vc3, pred_box,
      box_targets.reshape(B, M, 4))
    return (out_cls[0, 0], out_reg[0, 0])


# speedup vs baseline: 2.9903x; 1.5203x over previous
import functools

import jax
import jax.numpy as jnp
from jax.experimental import pallas as pl
from jax.experimental.pallas import tpu as pltpu

_C = 80
_THIRD = 1.0 / 3.0
_SUB = 16


def _tc_body(nblk, cls_ref, tc_ref, vc_ref, pb_ref, tb_ref,
             out_cls_ref, out_reg_ref, cvec, r_acc, n_acc):
    i = pl.program_id(0)

    @pl.when(i == 0)
    def _init():
        cvec[...] = jnp.zeros_like(cvec)
        r_acc[0] = 0.0
        n_acc[0] = 0.0

    x = cls_ref[...]
    tcol = tc_ref[0]
    vcol = vc_ref[0]

    e2 = jnp.exp(-x)
    a = 1.0 + e2
    p = 1.0 / a
    lg = jnp.log(a)
    sp = x + lg
    omp = e2 * p
    l0 = sp * p * p
    l1 = _THIRD * lg * omp * omp
    sub_iota = jax.lax.broadcasted_iota(
        jnp.int32, (128, _C), 1).astype(jnp.float32)
    csum = None
    for k in range(_SUB):
        tk = tcol[:, k:k + 1]
        vk = vcol[:, k:k + 1]
        l0k = l0[128 * k:128 * (k + 1), :]
        l1k = l1[128 * k:128 * (k + 1), :]
        flk = jnp.where(sub_iota == tk, l1k, l0k) * vk
        csum = flk if csum is None else csum + flk
    cvec[...] += jnp.sum(csum, axis=0)[None, :]

    b1 = pb_ref[...]
    b2 = tb_ref[...]
    b1x0, b1y0, b1x1, b1y1 = b1[0:1, :], b1[1:2, :], b1[2:3, :], b1[3:4, :]
    b2x0, b2y0, b2x1, b2y1 = b2[0:1, :], b2[1:2, :], b2[2:3, :], b2[3:4, :]
    area1 = (b1x1 - b1x0) * (b1y1 - b1y0)
    area2 = (b2x1 - b2x0) * (b2y1 - b2y0)
    iw = jnp.maximum(jnp.minimum(b1x1, b2x1) - jnp.maximum(b1x0, b2x0), 0.0)
    ih = jnp.maximum(jnp.minimum(b1y1, b2y1) - jnp.maximum(b1y0, b2y0), 0.0)
    inter = iw * ih
    union = area1 + area2 - inter
    cw = jnp.maximum(jnp.maximum(b1x1, b2x1) - jnp.minimum(b1x0, b2x0), 0.0)
    ch = jnp.maximum(jnp.maximum(b1y1, b2y1) - jnp.minimum(b1y0, b2y0), 0.0)
    areac = cw * ch
    giou = inter / union - (areac - union) / areac
    tlane = jnp.transpose(tcol, (1, 0))
    rsum = None
    csel = None
    for k in range(_SUB):
        fgk = tlane[k:k + 1, :] >= 0.0
        gk = giou[:, 128 * k:128 * (k + 1)][0:1, :]
        contrib = jnp.where(fgk, 1.0 - gk, 0.0)
        fsel = jnp.where(fgk, 1.0, 0.0)
        rsum = contrib if rsum is None else rsum + contrib
        csel = fsel if csel is None else csel + fsel
    r_acc[0] += jnp.sum(rsum)
    n_acc[0] += jnp.sum(csel)

    @pl.when(i == nblk - 1)
    def _fin():
        denom = jnp.maximum(n_acc[0], 1.0)
        out_cls_ref[...] = jnp.full((1, 1), 0.75 * jnp.sum(cvec[...]) / denom,
                                    jnp.float32)
        out_reg_ref[...] = jnp.full((1, 1), r_acc[0] / denom, jnp.float32)


def kernel(pred_cls, pred_box, mask, cls_targets, box_targets):
    B, M, C = pred_cls.shape
    N = B * M
    BLK = 2048
    G = N // BLK

    x = pred_cls.reshape(N, C)
    pb = pred_box.reshape(N, 4).T
    tb = box_targets.reshape(N, 4).T
    t = cls_targets.astype(jnp.float32)
    fg = (cls_targets >= 0) & (cls_targets != C)
    valid = (cls_targets >= 0) & jnp.logical_not(mask.reshape(N))
    tcmp = jnp.where(fg, t, -1.0)
    validf = valid.astype(jnp.float32)
    tc3 = tcmp.reshape(G, _SUB, 128).swapaxes(1, 2)
    vc3 = validf.reshape(G, _SUB, 128).swapaxes(1, 2)

    out_cls, out_reg = pl.pallas_call(
        functools.partial(_tc_body, G),
        grid=(G,),
        in_specs=[
            pl.BlockSpec((BLK, C), lambda i: (i, 0)),
            pl.BlockSpec((1, 128, _SUB), lambda i: (i, 0, 0)),
            pl.BlockSpec((1, 128, _SUB), lambda i: (i, 0, 0)),
            pl.BlockSpec((4, BLK), lambda i: (0, i)),
            pl.BlockSpec((4, BLK), lambda i: (0, i)),
        ],
        out_specs=[
            pl.BlockSpec((1, 1), lambda i: (0, 0)),
            pl.BlockSpec((1, 1), lambda i: (0, 0)),
        ],
        out_shape=[
            jax.ShapeDtypeStruct((1, 1), jnp.float32),
            jax.ShapeDtypeStruct((1, 1), jnp.float32),
        ],
        scratch_shapes=[
            pltpu.VMEM((1, C), jnp.float32),
            pltpu.SMEM((1,), jnp.float32),
            pltpu.SMEM((1,), jnp.float32),
        ],
    )(x, tc3, vc3, pb, tb)
    return (out_cls[0, 0], out_reg[0, 0])


# speedup vs baseline: 3.5574x; 1.1896x over previous
import functools

import jax
import jax.numpy as jnp
from jax.experimental import pallas as pl
from jax.experimental.pallas import tpu as pltpu

_C = 80
_THIRD = 1.0 / 3.0
_SUB = 64


def _tc_body(nblk, cls_ref, tc_ref, vc_ref, pb_ref, tb_ref,
             out_cls_ref, out_reg_ref, cvec, r_acc, n_acc):
    i = pl.program_id(0)

    @pl.when(i == 0)
    def _init():
        cvec[...] = jnp.zeros_like(cvec)
        r_acc[0] = 0.0
        n_acc[0] = 0.0

    x = cls_ref[...]
    tcol = tc_ref[0]
    vcol = vc_ref[0]

    e2 = jnp.exp(-x)
    a = 1.0 + e2
    p = 1.0 / a
    lg = jnp.log(a)
    sp = x + lg
    omp = e2 * p
    l0 = sp * p * p
    l1 = _THIRD * lg * omp * omp
    sub_iota = jax.lax.broadcasted_iota(
        jnp.int32, (128, _C), 1).astype(jnp.float32)
    csum = None
    for k in range(_SUB):
        tk = tcol[:, k:k + 1]
        vk = vcol[:, k:k + 1]
        l0k = l0[128 * k:128 * (k + 1), :]
        l1k = l1[128 * k:128 * (k + 1), :]
        flk = jnp.where(sub_iota == tk, l1k, l0k) * vk
        csum = flk if csum is None else csum + flk
    cvec[...] += jnp.sum(csum, axis=0)[None, :]

    b1 = pb_ref[...]
    b2 = tb_ref[...]
    b1x0, b1y0, b1x1, b1y1 = b1[0:1, :], b1[1:2, :], b1[2:3, :], b1[3:4, :]
    b2x0, b2y0, b2x1, b2y1 = b2[0:1, :], b2[1:2, :], b2[2:3, :], b2[3:4, :]
    area1 = (b1x1 - b1x0) * (b1y1 - b1y0)
    area2 = (b2x1 - b2x0) * (b2y1 - b2y0)
    iw = jnp.maximum(jnp.minimum(b1x1, b2x1) - jnp.maximum(b1x0, b2x0), 0.0)
    ih = jnp.maximum(jnp.minimum(b1y1, b2y1) - jnp.maximum(b1y0, b2y0), 0.0)
    inter = iw * ih
    union = area1 + area2 - inter
    cw = jnp.maximum(jnp.maximum(b1x1, b2x1) - jnp.minimum(b1x0, b2x0), 0.0)
    ch = jnp.maximum(jnp.maximum(b1y1, b2y1) - jnp.minimum(b1y0, b2y0), 0.0)
    areac = cw * ch
    giou = inter / union - (areac - union) / areac
    tlane = jnp.transpose(tcol, (1, 0))
    rsum = None
    csel = None
    for k in range(_SUB):
        fgk = tlane[k:k + 1, :] >= 0.0
        gk = giou[:, 128 * k:128 * (k + 1)][0:1, :]
        contrib = jnp.where(fgk, 1.0 - gk, 0.0)
        fsel = jnp.where(fgk, 1.0, 0.0)
        rsum = contrib if rsum is None else rsum + contrib
        csel = fsel if csel is None else csel + fsel
    r_acc[0] += jnp.sum(rsum)
    n_acc[0] += jnp.sum(csel)

    @pl.when(i == nblk - 1)
    def _fin():
        denom = jnp.maximum(n_acc[0], 1.0)
        out_cls_ref[...] = jnp.full((1, 1), 0.75 * jnp.sum(cvec[...]) / denom,
                                    jnp.float32)
        out_reg_ref[...] = jnp.full((1, 1), r_acc[0] / denom, jnp.float32)


def kernel(pred_cls, pred_box, mask, cls_targets, box_targets):
    B, M, C = pred_cls.shape
    N = B * M
    BLK = 8192
    G = N // BLK

    x = pred_cls.reshape(N, C)
    pb = pred_box.reshape(N, 4).T
    tb = box_targets.reshape(N, 4).T
    t = cls_targets.astype(jnp.float32)
    fg = (cls_targets >= 0) & (cls_targets != C)
    valid = (cls_targets >= 0) & jnp.logical_not(mask.reshape(N))
    tcmp = jnp.where(fg, t, -1.0)
    validf = valid.astype(jnp.float32)
    tc3 = tcmp.reshape(G, _SUB, 128).swapaxes(1, 2)
    vc3 = validf.reshape(G, _SUB, 128).swapaxes(1, 2)

    out_cls, out_reg = pl.pallas_call(
        functools.partial(_tc_body, G),
        grid=(G,),
        in_specs=[
            pl.BlockSpec((BLK, C), lambda i: (i, 0)),
            pl.BlockSpec((1, 128, _SUB), lambda i: (i, 0, 0)),
            pl.BlockSpec((1, 128, _SUB), lambda i: (i, 0, 0)),
            pl.BlockSpec((4, BLK), lambda i: (0, i)),
            pl.BlockSpec((4, BLK), lambda i: (0, i)),
        ],
        out_specs=[
            pl.BlockSpec((1, 1), lambda i: (0, 0)),
            pl.BlockSpec((1, 1), lambda i: (0, 0)),
        ],
        out_shape=[
            jax.ShapeDtypeStruct((1, 1), jnp.float32),
            jax.ShapeDtypeStruct((1, 1), jnp.float32),
        ],
        scratch_shapes=[
            pltpu.VMEM((1, C), jnp.float32),
            pltpu.SMEM((1,), jnp.float32),
            pltpu.SMEM((1,), jnp.float32),
        ],
    )(x, tc3, vc3, pb, tb)
    return (out_cls[0, 0], out_reg[0, 0])


# speedup vs baseline: 3.5807x; 1.0066x over previous
import functools

import jax
import jax.numpy as jnp
from jax.experimental import pallas as pl
from jax.experimental.pallas import tpu as pltpu

_C = 80
_THIRD = 1.0 / 3.0
_SUB = 128


def _tc_body(nblk, cls_ref, tc_ref, vc_ref, pb_ref, tb_ref,
             out_cls_ref, out_reg_ref, cvec, r_acc, n_acc):
    i = pl.program_id(0)

    @pl.when(i == 0)
    def _init():
        cvec[...] = jnp.zeros_like(cvec)
        r_acc[0] = 0.0
        n_acc[0] = 0.0

    x = cls_ref[...]
    tcol = tc_ref[0]
    vcol = vc_ref[0]

    e2 = jnp.exp(-x)
    a = 1.0 + e2
    p = 1.0 / a
    lg = jnp.log(a)
    sp = x + lg
    omp = e2 * p
    l0 = sp * p * p
    l1 = _THIRD * lg * omp * omp
    sub_iota = jax.lax.broadcasted_iota(
        jnp.int32, (128, _C), 1).astype(jnp.float32)
    csum = None
    for k in range(_SUB):
        tk = tcol[:, k:k + 1]
        vk = vcol[:, k:k + 1]
        l0k = l0[128 * k:128 * (k + 1), :]
        l1k = l1[128 * k:128 * (k + 1), :]
        flk = jnp.where(sub_iota == tk, l1k, l0k) * vk
        csum = flk if csum is None else csum + flk
    cvec[...] += jnp.sum(csum, axis=0)[None, :]

    b1 = pb_ref[...]
    b2 = tb_ref[...]
    b1x0, b1y0, b1x1, b1y1 = b1[0:1, :], b1[1:2, :], b1[2:3, :], b1[3:4, :]
    b2x0, b2y0, b2x1, b2y1 = b2[0:1, :], b2[1:2, :], b2[2:3, :], b2[3:4, :]
    area1 = (b1x1 - b1x0) * (b1y1 - b1y0)
    area2 = (b2x1 - b2x0) * (b2y1 - b2y0)
    iw = jnp.maximum(jnp.minimum(b1x1, b2x1) - jnp.maximum(b1x0, b2x0), 0.0)
    ih = jnp.maximum(jnp.minimum(b1y1, b2y1) - jnp.maximum(b1y0, b2y0), 0.0)
    inter = iw * ih
    union = area1 + area2 - inter
    cw = jnp.maximum(jnp.maximum(b1x1, b2x1) - jnp.minimum(b1x0, b2x0), 0.0)
    ch = jnp.maximum(jnp.maximum(b1y1, b2y1) - jnp.minimum(b1y0, b2y0), 0.0)
    areac = cw * ch
    giou = inter / union - (areac - union) / areac
    tlane = jnp.transpose(tcol, (1, 0))
    rsum = None
    csel = None
    for k in range(_SUB):
        fgk = tlane[k:k + 1, :] >= 0.0
        gk = giou[:, 128 * k:128 * (k + 1)][0:1, :]
        contrib = jnp.where(fgk, 1.0 - gk, 0.0)
        fsel = jnp.where(fgk, 1.0, 0.0)
        rsum = contrib if rsum is None else rsum + contrib
        csel = fsel if csel is None else csel + fsel
    r_acc[0] += jnp.sum(rsum)
    n_acc[0] += jnp.sum(csel)

    @pl.when(i == nblk - 1)
    def _fin():
        denom = jnp.maximum(n_acc[0], 1.0)
        out_cls_ref[...] = jnp.full((1, 1), 0.75 * jnp.sum(cvec[...]) / denom,
                                    jnp.float32)
        out_reg_ref[...] = jnp.full((1, 1), r_acc[0] / denom, jnp.float32)


def kernel(pred_cls, pred_box, mask, cls_targets, box_targets):
    B, M, C = pred_cls.shape
    N = B * M
    BLK = 16384
    G = N // BLK

    x = pred_cls.reshape(N, C)
    pb = pred_box.reshape(N, 4).T
    tb = box_targets.reshape(N, 4).T
    t = cls_targets.astype(jnp.float32)
    fg = (cls_targets >= 0) & (cls_targets != C)
    valid = (cls_targets >= 0) & jnp.logical_not(mask.reshape(N))
    tcmp = jnp.where(fg, t, -1.0)
    validf = valid.astype(jnp.float32)
    tc3 = tcmp.reshape(G, _SUB, 128).swapaxes(1, 2)
    vc3 = validf.reshape(G, _SUB, 128).swapaxes(1, 2)

    out_cls, out_reg = pl.pallas_call(
        functools.partial(_tc_body, G),
        grid=(G,),
        in_specs=[
            pl.BlockSpec((BLK, C), lambda i: (i, 0)),
            pl.BlockSpec((1, 128, _SUB), lambda i: (i, 0, 0)),
            pl.BlockSpec((1, 128, _SUB), lambda i: (i, 0, 0)),
            pl.BlockSpec((4, BLK), lambda i: (0, i)),
            pl.BlockSpec((4, BLK), lambda i: (0, i)),
        ],
        out_specs=[
            pl.BlockSpec((1, 1), lambda i: (0, 0)),
            pl.BlockSpec((1, 1), lambda i: (0, 0)),
        ],
        out_shape=[
            jax.ShapeDtypeStruct((1, 1), jnp.float32),
            jax.ShapeDtypeStruct((1, 1), jnp.float32),
        ],
        scratch_shapes=[
            pltpu.VMEM((1, C), jnp.float32),
            pltpu.SMEM((1,), jnp.float32),
            pltpu.SMEM((1,), jnp.float32),
        ],
    )(x, tc3, vc3, pb, tb)
    return (out_cls[0, 0], out_reg[0, 0])
